# baseline (speedup 1.0000x reference)
import functools

import jax
import jax.numpy as jnp
from jax.experimental import pallas as pl

B_N = 9


def _phase1_body(nsteps, fv_ref, mb_ref, out_ref):
    i = pl.program_id(0)
    fv = fv_ref[...]
    mb = mb_ref[...]
    prod = jax.lax.dot_general(fv, mb, (((1,), (1,)), ((), ())))
    fvn = jnp.sum(fv * fv, axis=1, keepdims=True)
    mbn = jax.lax.dot_general(jnp.ones((1, fv.shape[1]), fv.dtype), mb * mb,
                              (((1,), (1,)), ((), ())))
    d = fvn + mbn - 2.0 * prod
    part = jnp.min(d, axis=1, keepdims=True)

    @pl.when(i == 0)
    def _():
        out_ref[...] = part

    @pl.when(i > 0)
    def _():
        out_ref[...] = jnp.minimum(out_ref[...], part)

    @pl.when(i == nsteps - 1)
    def _():
        out_ref[...] = jnp.sqrt(jnp.maximum(out_ref[...], 0.0))


def _phase2_body(batch, hw, fv_ref, mb_ref, pix_ref, img_ref):
    fv = fv_ref[...]
    mb = mb_ref[...]
    s = pix_ref[...]
    q = fv.shape[0]

    row_iota = jax.lax.broadcasted_iota(jnp.int32, (q, 1), 0)
    rows = []
    for b in range(batch):
        in_b = (row_iota >= b * hw) & (row_iota < (b + 1) * hw)
        sb = jnp.where(in_b, s, -jnp.inf)
        m = jnp.max(sb)
        idx = jnp.min(jnp.where(sb == m, row_iota, jnp.int32(2 ** 30)))
        onehot = (row_iota == idx).astype(fv.dtype)
        rows.append(jnp.sum(fv * onehot, axis=0, keepdims=True))
    sel = jnp.concatenate(rows, axis=0)

    prod = jax.lax.dot_general(sel, mb, (((1,), (1,)), ((), ())))
    seln = jnp.sum(sel * sel, axis=1, keepdims=True)
    mbn = jax.lax.dot_general(jnp.ones((1, mb.shape[1]), mb.dtype), mb * mb,
                              (((1,), (1,)), ((), ())))
    d = jnp.maximum(seln + mbn - 2.0 * prod, 0.0)

    col_iota = jax.lax.broadcasted_iota(jnp.int32, d.shape, 1)
    mins = []
    for _ in range(B_N):
        mn = jnp.min(d, axis=1, keepdims=True)
        mins.append(mn)
        amn = jnp.min(jnp.where(d == mn, col_iota, jnp.int32(2 ** 30)),
                      axis=1, keepdims=True)
        d = jnp.where(col_iota == amn, jnp.inf, d)
    sd = jnp.sqrt(jnp.concatenate(mins, axis=1))

    mx = jnp.max(sd, axis=1, keepdims=True)
    e = jnp.exp(sd - mx)
    p0 = e[:, 0:1] / jnp.sum(e, axis=1, keepdims=True)
    img_ref[...] = sd[:, 0:1] * (1.0 - p0)


def kernel(feature_batch, mb):
    batch, height, width, channels = feature_batch.shape
    hw = height * width
    q = batch * hw
    m = mb.shape[0]
    fv = jnp.reshape(feature_batch, (q, channels))

    tile = 1024
    nsteps = m // tile
    pix = pl.pallas_call(
        functools.partial(_phase1_body, nsteps),
        grid=(nsteps,),
        in_specs=[
            pl.BlockSpec((q, channels), lambda i: (0, 0)),
            pl.BlockSpec((tile, channels), lambda i: (i, 0)),
        ],
        out_specs=pl.BlockSpec((q, 1), lambda i: (0, 0)),
        out_shape=jax.ShapeDtypeStruct((q, 1), fv.dtype),
    )(fv, mb)

    img = pl.pallas_call(
        functools.partial(_phase2_body, batch, hw),
        out_shape=jax.ShapeDtypeStruct((batch, 1), fv.dtype),
    )(fv, mb, pix)

    pixel_scores = jnp.reshape(pix, (batch, 1, height, width))
    image_scores = jnp.reshape(img, (batch,))
    return (pixel_scores, image_scores)
